# Initial kernel scaffold; baseline (speedup 1.0000x reference)
import jax
import jax.numpy as jnp
from jax.experimental import pallas as pl


def kernel(x_tab, x_gnn, edge_attr, W_in, b_in, W_o, b_o, W_ff1, b_ff1, W_ff2, b_ff2, g_n1, b_n1, g_n2, b_n2, g_tab, b_tab, W_e, b_e, W_pre, b_pre, W_post, b_post, W_lin, b_lin, g_bn, b_bn, W_eu1, b_eu1, W_eu2, b_eu2, edge_index):
    raise NotImplementedError("write your pallas kernel here")


# speedup vs baseline: 1.2549x; 1.2549x over previous
import functools
import math

import jax
import jax.numpy as jnp
import numpy as np
from jax.experimental import pallas as pl
from jax.experimental.pallas import tpu as pltpu

AVG_LOG = float(np.log(33.0))
NHEAD = 8


def _ln(x, g, b, eps=1e-5):
    m = jnp.mean(x, axis=-1, keepdims=True)
    v = jnp.mean((x - m) * (x - m), axis=-1, keepdims=True)
    return (x - m) * jax.lax.rsqrt(v + eps) * g + b


def _tab_body(x_ref, W_in_ref, b_in_ref, W_o_ref, b_o_ref, W_ff1_ref, b_ff1_ref,
              W_ff2_ref, b_ff2_ref, g1_ref, b1_ref, g2_ref, b2_ref, gt_ref,
              bt_ref, o_ref):
    R, S, D = x_ref.shape
    dh = D // NHEAD
    x2 = x_ref[...].reshape(R * S, D)
    qkv = jnp.dot(x2, W_in_ref[...].T, preferred_element_type=jnp.float32)
    qkv = qkv + b_in_ref[...]
    q, k, v = qkv[:, :D], qkv[:, D:2 * D], qkv[:, 2 * D:]
    k3 = k.reshape(R, S, D)
    v3 = v.reshape(R, S, D)
    q3 = q.reshape(R, S, D)
    lane = jax.lax.broadcasted_iota(jnp.int32, (D, NHEAD), 0)
    head = jax.lax.broadcasted_iota(jnp.int32, (D, NHEAD), 1)
    H = (lane // dh == head).astype(jnp.float32)
    inv_sqrt = 1.0 / math.sqrt(dh)
    outs = []
    for i in range(S):
        qi = q3[:, i:i + 1, :]
        p = (qi * k3).reshape(R * S, D)
        s = jnp.dot(p, H, preferred_element_type=jnp.float32) * inv_sqrt
        s3 = s.reshape(R, S, NHEAD)
        mx = jnp.max(s3, axis=1, keepdims=True)
        e = jnp.exp(s3 - mx)
        z = jnp.sum(e, axis=1, keepdims=True)
        a = e / z
        aexp = jnp.dot(a.reshape(R * S, NHEAD), H.T,
                       preferred_element_type=jnp.float32)
        o_i = jnp.sum((aexp * v).reshape(R, S, D), axis=1, keepdims=True)
        outs.append(o_i)
    o = jnp.concatenate(outs, axis=1).reshape(R * S, D)
    o = jnp.dot(o, W_o_ref[...].T, preferred_element_type=jnp.float32) + b_o_ref[...]
    t = _ln(x2 + o, g1_ref[...], b1_ref[...])
    ff = jnp.maximum(
        jnp.dot(t, W_ff1_ref[...].T, preferred_element_type=jnp.float32)
        + b_ff1_ref[...], 0.0)
    ff = jnp.dot(ff, W_ff2_ref[...].T, preferred_element_type=jnp.float32) + b_ff2_ref[...]
    t = _ln(t + ff, g2_ref[...], b2_ref[...])
    t = _ln(t, gt_ref[...], bt_ref[...])
    o_ref[...] = t.reshape(R, S, D)


def _tab_branch(x_tab, W_in, b_in, W_o, b_o, W_ff1, b_ff1, W_ff2, b_ff2,
                g_n1, b_n1, g_n2, b_n2, g_tab, b_tab):
    N, S, D = x_tab.shape
    R = 400
    NP = ((N + R - 1) // R) * R
    xp = jnp.pad(x_tab, ((0, NP - N), (0, 0), (0, 0)))
    row = lambda r: (1, 384)
    full = lambda arr: pl.BlockSpec(arr.shape, lambda i: (0,) * arr.ndim)
    w_specs = []
    ws = [W_in, b_in.reshape(1, -1), W_o, b_o.reshape(1, -1),
          W_ff1, b_ff1.reshape(1, -1), W_ff2, b_ff2.reshape(1, -1),
          g_n1.reshape(1, -1), b_n1.reshape(1, -1), g_n2.reshape(1, -1),
          b_n2.reshape(1, -1), g_tab.reshape(1, -1), b_tab.reshape(1, -1)]
    for w in ws:
        w_specs.append(full(w))
    out = pl.pallas_call(
        _tab_body,
        grid=(NP // R,),
        in_specs=[pl.BlockSpec((R, S, D), lambda i: (i, 0, 0))] + w_specs,
        out_specs=pl.BlockSpec((R, S, D), lambda i: (i, 0, 0)),
        out_shape=jax.ShapeDtypeStruct((NP, S, D), jnp.float32),
    )(xp, *ws)
    return out[:N]


def _edge_mm_body(ea_ref, W_ref, o_ref):
    o_ref[...] = jnp.dot(ea_ref[...], W_ref[...].T,
                         preferred_element_type=jnp.float32)


def _edge_matmul(edge_attr, W):
    E, D = edge_attr.shape
    BE = 1280
    return pl.pallas_call(
        _edge_mm_body,
        grid=(E // BE,),
        in_specs=[pl.BlockSpec((BE, D), lambda i: (i, 0)),
                  pl.BlockSpec(W.shape, lambda i: (0, 0))],
        out_specs=pl.BlockSpec((BE, D), lambda i: (i, 0)),
        out_shape=jax.ShapeDtypeStruct((E, D), jnp.float32),
    )(edge_attr, W)


def _post_a_body(x_ref, sum_ref, ssq_ref, mx_ref, mn_ref, deg_ref, Wpi_ref,
                 ba_ref, W0_ref, Wa_ref, Wb_ref, Wc_ref, bpost_ref, Wlin_ref,
                 blin_ref, out_ref, bs_ref, bq_ref):
    x = x_ref[...]
    deg = deg_ref[...]
    degc = jnp.maximum(deg, 1.0)
    A = jnp.dot(x, Wpi_ref[...].T, preferred_element_type=jnp.float32) + ba_ref[...]
    s = sum_ref[...]
    ssq = ssq_ref[...]
    mean = (s + deg * A) / degc
    mean2 = (ssq + 2.0 * A * s + deg * A * A) / degc
    std = jnp.sqrt(jnp.maximum(mean2 - mean * mean, 0.0) + 1e-5)
    has = deg > 0.0
    mx = jnp.where(has, mx_ref[...] + A, 0.0)
    mn = jnp.where(has, mn_ref[...] + A, 0.0)
    lg = jnp.log(degc + 1.0)
    amp = lg * (1.0 / AVG_LOG)
    att = AVG_LOG / lg
    agg = jnp.concatenate([mean, mx, mn, std], axis=1)
    out = (jnp.dot(x, W0_ref[...].T, preferred_element_type=jnp.float32)
           + jnp.dot(agg, Wa_ref[...].T, preferred_element_type=jnp.float32)
           + jnp.dot(agg * amp, Wb_ref[...].T, preferred_element_type=jnp.float32)
           + jnp.dot(agg * att, Wc_ref[...].T, preferred_element_type=jnp.float32)
           + bpost_ref[...])
    out = jnp.dot(out, Wlin_ref[...].T, preferred_element_type=jnp.float32) + blin_ref[...]
    out_ref[...] = out
    @pl.when(pl.program_id(0) == 0)
    def _init():
        bs_ref[...] = jnp.zeros_like(bs_ref)
        bq_ref[...] = jnp.zeros_like(bq_ref)
    bs_ref[...] += jnp.sum(out, axis=0, keepdims=True)
    bq_ref[...] += jnp.sum(out * out, axis=0, keepdims=True)


def _post_b_body(x_ref, out_ref, bs_ref, bq_ref, gbn_ref, bbn_ref, Weus_ref,
                 Weud_ref, beu1_ref, xout_ref, P_ref, Q_ref, *, n_rows):
    x = x_ref[...]
    out = out_ref[...]
    bm = bs_ref[...] * (1.0 / n_rows)
    bv = bq_ref[...] * (1.0 / n_rows) - bm * bm
    bn = (out - bm) * jax.lax.rsqrt(bv + 1e-5) * gbn_ref[...] + bbn_ref[...]
    xo = (x + jnp.maximum(bn, 0.0)) * 0.5
    xout_ref[...] = xo
    P_ref[...] = jnp.dot(xo, Weus_ref[...].T, preferred_element_type=jnp.float32)
    Q_ref[...] = (jnp.dot(xo, Weud_ref[...].T, preferred_element_type=jnp.float32)
                  + beu1_ref[...])


def _post_pass(x_gnn, sum_u, ssq_u, mx_u, mn_u, deg, Wpre, b_pre, W_e, b_e,
               W_post, b_post, W_lin, b_lin, g_bn, b_bn, W_eu1, b_eu1):
    Nn, D = x_gnn.shape
    BR = 1000
    Wpi = Wpre[:, :D]
    Wpe = Wpre[:, 2 * D:]
    bias_a = (b_pre + Wpe @ b_e).reshape(1, D)
    W0 = W_post[:, :D]
    Wa = W_post[:, D:5 * D]
    Wb = W_post[:, 5 * D:9 * D]
    Wc = W_post[:, 9 * D:13 * D]
    Weus = W_eu1[:, :D]
    Weud = W_eu1[:, D:2 * D]
    row = lambda a: pl.BlockSpec((BR, a.shape[1]), lambda i: (i, 0))
    full = lambda a: pl.BlockSpec(a.shape, lambda i: (0, 0))
    args_a = [x_gnn, sum_u, ssq_u, mx_u, mn_u, deg.reshape(Nn, 1)]
    w_a = [Wpi, bias_a, W0, Wa, Wb, Wc, b_post.reshape(1, D), W_lin,
           b_lin.reshape(1, D)]
    out, bs, bq = pl.pallas_call(
        _post_a_body,
        grid=(Nn // BR,),
        in_specs=[row(a) for a in args_a] + [full(w) for w in w_a],
        out_specs=[pl.BlockSpec((BR, D), lambda i: (i, 0)),
                   pl.BlockSpec((1, D), lambda i: (0, 0)),
                   pl.BlockSpec((1, D), lambda i: (0, 0))],
        out_shape=[jax.ShapeDtypeStruct((Nn, D), jnp.float32),
                   jax.ShapeDtypeStruct((1, D), jnp.float32),
                   jax.ShapeDtypeStruct((1, D), jnp.float32)],
    )(*args_a, *w_a)
    w_b = [bs, bq, g_bn.reshape(1, D), b_bn.reshape(1, D), Weus, Weud,
           b_eu1.reshape(1, D)]
    xo, P, Q = pl.pallas_call(
        functools.partial(_post_b_body, n_rows=float(Nn)),
        grid=(Nn // BR,),
        in_specs=[row(x_gnn), row(out)] + [full(w) for w in w_b],
        out_specs=[pl.BlockSpec((BR, D), lambda i: (i, 0))] * 3,
        out_shape=[jax.ShapeDtypeStruct((Nn, D), jnp.float32)] * 3,
    )(x_gnn, out, *w_b)
    return xo, P, Q


def _edge_final_body(ea_ref, G_ref, Weue_ref, Weu2_ref, beu2_ref, o_ref):
    ea = ea_ref[...]
    h = jnp.maximum(
        G_ref[...] + jnp.dot(ea, Weue_ref[...].T, preferred_element_type=jnp.float32),
        0.0)
    o_ref[...] = ea + jnp.dot(h, Weu2_ref[...].T,
                              preferred_element_type=jnp.float32) + beu2_ref[...]


def _edge_final(edge_attr, G, W_eu1, W_eu2, b_eu2):
    E, D = edge_attr.shape
    Weue = W_eu1[:, 2 * D:]
    Weu2h = W_eu2 * 0.5
    beu2h = (b_eu2 * 0.5).reshape(1, D)
    BE = 1280
    return pl.pallas_call(
        _edge_final_body,
        grid=(E // BE,),
        in_specs=[pl.BlockSpec((BE, D), lambda i: (i, 0)),
                  pl.BlockSpec((BE, D), lambda i: (i, 0)),
                  pl.BlockSpec(Weue.shape, lambda i: (0, 0)),
                  pl.BlockSpec(Weu2h.shape, lambda i: (0, 0)),
                  pl.BlockSpec(beu2h.shape, lambda i: (0, 0))],
        out_specs=pl.BlockSpec((BE, D), lambda i: (i, 0)),
        out_shape=jax.ShapeDtypeStruct((E, D), jnp.float32),
    )(edge_attr, G, Weue, Weu2h, beu2h)


def kernel(x_tab, x_gnn, edge_attr, W_in, b_in, W_o, b_o, W_ff1, b_ff1, W_ff2,
           b_ff2, g_n1, b_n1, g_n2, b_n2, g_tab, b_tab, W_e, b_e, W_pre, b_pre,
           W_post, b_post, W_lin, b_lin, g_bn, b_bn, W_eu1, b_eu1, W_eu2,
           b_eu2, edge_index):
    Nn, D = x_gnn.shape
    E = edge_attr.shape[0]
    src = edge_index[0]
    dst = edge_index[1]

    x_tab_out = _tab_branch(x_tab, W_in, b_in, W_o, b_o, W_ff1, b_ff1, W_ff2,
                            b_ff2, g_n1, b_n1, g_n2, b_n2, g_tab, b_tab)

    Wpj = W_pre[:, D:2 * D]
    Wce = W_pre[:, 2 * D:] @ W_e
    B = x_gnn @ Wpj.T
    C = _edge_matmul(edge_attr, Wce)

    u = B[src] + C
    ones = jnp.ones((E,), jnp.float32)
    deg = jax.ops.segment_sum(ones, dst, num_segments=Nn)
    sum_u = jax.ops.segment_sum(u, dst, num_segments=Nn)
    ssq_u = jax.ops.segment_sum(u * u, dst, num_segments=Nn)
    mx_u = jax.ops.segment_max(u, dst, num_segments=Nn)
    mx_u = jnp.where(jnp.isfinite(mx_u), mx_u, 0.0)
    mn_u = jax.ops.segment_min(u, dst, num_segments=Nn)
    mn_u = jnp.where(jnp.isfinite(mn_u), mn_u, 0.0)

    x_gnn_out, P, Q = _post_pass(x_gnn, sum_u, ssq_u, mx_u, mn_u, deg, W_pre,
                                 b_pre, W_e, b_e, W_post, b_post, W_lin, b_lin,
                                 g_bn, b_bn, W_eu1, b_eu1)

    G = P[src] + Q[dst]
    edge_out = _edge_final(edge_attr, G, W_eu1, W_eu2, b_eu2)

    return (x_tab_out, x_gnn_out, edge_out)
